# baseline (speedup 1.0000x reference)
import functools

import jax
import jax.numpy as jnp
from jax import lax
from jax.experimental import pallas as pl
from jax.experimental.pallas import tpu as pltpu
from jax.experimental.pallas import tpu_sc as plsc

_B = 4096
_L = 200
_D = 64
_NC = 2
_NS = 16
_NW = _NC * _NS
_RPW = _B // _NW
_SIZES = (128, 64, 32, 16, 8, 4, 2, 1)


def _emb_body(idx_hbm, seq_hbm, table_hbm, zeros_hbm, ones_hbm,
              out_hbm, mask_hbm,
              idx_v, seq_v, rows_v, zeros_v, ones_v,
              sem_g, sem_w0, sem_w1):
    wid = lax.axis_index("s") * _NC + lax.axis_index("c")
    base = wid * _RPW
    pltpu.sync_copy(seq_hbm.at[pl.ds(base, _RPW)], seq_v.at[pl.ds(0, _RPW)])
    pltpu.sync_copy(zeros_hbm, zeros_v)
    pltpu.sync_copy(ones_hbm, ones_v)

    def do_row(i, slot, sem_w, j):
        b = base + i
        row0 = b * _L

        @pl.when(j >= 1)
        def _():
            pltpu.make_async_copy(table_hbm.at[pl.ds(0, _L), :],
                                  rows_v.at[slot], sem_w).wait()
            pltpu.make_async_copy(table_hbm.at[pl.ds(0, _L), :],
                                  rows_v.at[slot], sem_w).wait()

        pltpu.sync_copy(idx_hbm.at[pl.ds(row0, _L)], idx_v)
        g1 = pltpu.async_copy(table_hbm.at[idx_v.at[pl.ds(0, 128)]],
                              rows_v.at[slot, pl.ds(0, 128), :], sem_g)
        g2 = pltpu.async_copy(table_hbm.at[idx_v.at[pl.ds(128, _L - 128)]],
                              rows_v.at[slot, pl.ds(128, _L - 128), :], sem_g)

        ln = seq_v[pl.ds(i, 16)][0]
        rem = _L - ln

        off = 0
        soff = ln
        for s in _SIZES:
            pbit = (ln & s) != 0
            sbit = (rem & s) != 0

            @pl.when(pbit)
            def _(off=off, s=s):
                pltpu.async_copy(ones_v.at[pl.ds(0, s), :],
                                 mask_hbm.at[pl.ds(row0 + off, s), :], sem_w)

            @pl.when(sbit)
            def _(soff=soff, s=s):
                pltpu.async_copy(zeros_v.at[pl.ds(0, s), :],
                                 out_hbm.at[pl.ds(row0 + soff, s), :], sem_w)
                pltpu.async_copy(zeros_v.at[pl.ds(0, s), :],
                                 mask_hbm.at[pl.ds(row0 + soff, s), :], sem_w)

            off = off + s * pbit.astype(jnp.int32)
            soff = soff + s * sbit.astype(jnp.int32)

        g1.wait()
        g2.wait()

        off = 0
        for s in _SIZES:
            pbit = (ln & s) != 0

            @pl.when(pbit)
            def _(off=off, s=s):
                pltpu.async_copy(rows_v.at[slot, pl.ds(off, s), :],
                                 out_hbm.at[pl.ds(row0 + off, s), :], sem_w)

            off = off + s * pbit.astype(jnp.int32)

    def body(j, carry):
        do_row(2 * j, 0, sem_w0, j)
        do_row(2 * j + 1, 1, sem_w1, j)
        return carry

    lax.fori_loop(0, _RPW // 2, body, 0)

    for sem_w, slot in ((sem_w0, 0), (sem_w1, 1)):
        pltpu.make_async_copy(table_hbm.at[pl.ds(0, _L), :],
                              rows_v.at[slot], sem_w).wait()
        pltpu.make_async_copy(table_hbm.at[pl.ds(0, _L), :],
                              rows_v.at[slot], sem_w).wait()


@functools.partial(jax.jit, static_argnames=())
def _emb_call(idx_flat, seq, table, zeros, ones):
    mesh = plsc.VectorSubcoreMesh(core_axis_name="c", subcore_axis_name="s",
                                  num_cores=_NC, num_subcores=_NS)
    fn = pl.kernel(
        _emb_body,
        out_type=(jax.ShapeDtypeStruct((_B * _L, _D), jnp.float32),
                  jax.ShapeDtypeStruct((_B * _L, _D), jnp.float32)),
        mesh=mesh,
        scratch_types=[
            pltpu.VMEM((_L,), jnp.int32),
            pltpu.VMEM((_RPW + 16,), jnp.int32),
            pltpu.VMEM((2, _L, _D), jnp.float32),
            pltpu.VMEM((128, _D), jnp.float32),
            pltpu.VMEM((128, _D), jnp.float32),
            pltpu.SemaphoreType.DMA,
            pltpu.SemaphoreType.DMA,
            pltpu.SemaphoreType.DMA,
        ],
        compiler_params=pltpu.CompilerParams(use_tc_tiling_on_sc=False),
    )
    return fn(idx_flat, seq, table, zeros, ones)


def kernel(indices, seq_lens, table):
    idx_flat = indices.reshape(_B * _L).astype(jnp.int32)
    seq = seq_lens.astype(jnp.int32)
    zeros = jnp.zeros((128, _D), jnp.float32)
    ones = jnp.ones((128, _D), jnp.float32)
    out, mask = _emb_call(idx_flat, seq, table, zeros, ones)
    return out.reshape(_B, _L, _D), mask.reshape(_B, _L, _D)


# speedup vs baseline: 1.2580x; 1.2580x over previous
import functools

import jax
import jax.numpy as jnp
from jax import lax
from jax.experimental import pallas as pl
from jax.experimental.pallas import tpu as pltpu
from jax.experimental.pallas import tpu_sc as plsc

_B = 4096
_L = 200
_D = 64
_NC = 2
_NS = 16
_NW = _NC * _NS
_RPW = _B // _NW
_SIZES = (128, 64, 32, 16, 8, 4, 2, 1)


def _emb_body(idx_hbm, seq_hbm, table_hbm, zeros_hbm,
              out_hbm,
              idx_v, seq_v, rows_v, zeros_v,
              sem_g, sem_w0, sem_w1):
    wid = lax.axis_index("s") * _NC + lax.axis_index("c")
    base = wid * _RPW
    pltpu.sync_copy(seq_hbm.at[pl.ds(base, _RPW)], seq_v.at[pl.ds(0, _RPW)])
    pltpu.sync_copy(zeros_hbm, zeros_v)

    def do_row(i, slot, sem_w, j):
        b = base + i
        row0 = b * _L

        @pl.when(j >= 1)
        def _():
            pltpu.make_async_copy(table_hbm.at[pl.ds(0, _L), :],
                                  rows_v.at[slot], sem_w).wait()

        pltpu.sync_copy(idx_hbm.at[pl.ds(row0, _L)], idx_v)
        g1 = pltpu.async_copy(table_hbm.at[idx_v.at[pl.ds(0, 128)]],
                              rows_v.at[slot, pl.ds(0, 128), :], sem_g)
        g2 = pltpu.async_copy(table_hbm.at[idx_v.at[pl.ds(128, _L - 128)]],
                              rows_v.at[slot, pl.ds(128, _L - 128), :], sem_g)

        ln = seq_v[pl.ds(i, 16)][0]
        rem = _L - ln

        soff = ln
        for s in _SIZES:
            sbit = (rem & s) != 0

            @pl.when(sbit)
            def _(soff=soff, s=s):
                pltpu.async_copy(zeros_v.at[pl.ds(0, s), :],
                                 out_hbm.at[pl.ds(row0 + soff, s), :], sem_w)

            soff = soff + s * sbit.astype(jnp.int32)

        g1.wait()
        g2.wait()

        off = 0
        for s in _SIZES:
            pbit = (ln & s) != 0

            @pl.when(pbit)
            def _(off=off, s=s):
                pltpu.async_copy(rows_v.at[slot, pl.ds(off, s), :],
                                 out_hbm.at[pl.ds(row0 + off, s), :], sem_w)

            off = off + s * pbit.astype(jnp.int32)

    def body(j, carry):
        do_row(2 * j, 0, sem_w0, j)
        do_row(2 * j + 1, 1, sem_w1, j)
        return carry

    lax.fori_loop(0, _RPW // 2, body, 0)

    for sem_w, slot in ((sem_w0, 0), (sem_w1, 1)):
        pltpu.make_async_copy(table_hbm.at[pl.ds(0, _L), :],
                              rows_v.at[slot], sem_w).wait()


@jax.jit
def _emb_call(idx_flat, seq, table, zeros):
    mesh = plsc.VectorSubcoreMesh(core_axis_name="c", subcore_axis_name="s",
                                  num_cores=_NC, num_subcores=_NS)
    fn = pl.kernel(
        _emb_body,
        out_type=jax.ShapeDtypeStruct((_B * _L, _D), jnp.float32),
        mesh=mesh,
        scratch_types=[
            pltpu.VMEM((_L,), jnp.int32),
            pltpu.VMEM((_RPW + 16,), jnp.int32),
            pltpu.VMEM((2, _L, _D), jnp.float32),
            pltpu.VMEM((128, _D), jnp.float32),
            pltpu.SemaphoreType.DMA,
            pltpu.SemaphoreType.DMA,
            pltpu.SemaphoreType.DMA,
        ],
        compiler_params=pltpu.CompilerParams(use_tc_tiling_on_sc=False),
    )
    return fn(idx_flat, seq, table, zeros)


def kernel(indices, seq_lens, table):
    idx_flat = indices.reshape(_B * _L).astype(jnp.int32)
    seq = seq_lens.astype(jnp.int32)
    zeros = jnp.zeros((128, _D), jnp.float32)
    out = _emb_call(idx_flat, seq, table, zeros)
    mask = (jnp.arange(_L, dtype=jnp.int32)[None, :]
            < seq_lens.astype(jnp.int32)[:, None]).astype(table.dtype)
    lengths = jnp.broadcast_to(mask[:, :, None], (_B, _L, _D))
    return out.reshape(_B, _L, _D), lengths
